# Initial kernel scaffold; baseline (speedup 1.0000x reference)
import jax
import jax.numpy as jnp
from jax.experimental import pallas as pl


def kernel(pc):
    raise NotImplementedError("write your pallas kernel here")


# speedup vs baseline: 87.0947x; 87.0947x over previous
import dataclasses

import jax
import jax.numpy as jnp
from jax import lax
from jax.experimental import pallas as pl
from jax.experimental.pallas import tpu as pltpu
from jax.experimental.pallas import tpu_sc as plsc

HEIGHT = 512
WIDTH = 512
INTENSITY = 255.0

N_POINTS = 2_000_000
N_FLOATS = 3 * N_POINTS

CHUNK_PTS = 640
CHUNK_F = 3 * CHUNK_PTS
N_CHUNKS = N_POINTS // CHUNK_PTS
N_WORKERS = 32
ITERS = -(-N_CHUNKS // N_WORKERS)
BATCHES = CHUNK_PTS // 128
N_BINS = HEIGHT * WIDTH
BINS_PER_TILE = N_BINS // 16


def _minmax_tc(pcv):
    def body(x_ref, o_ref):
        o_ref[0:1, :] = jnp.min(x_ref[...], axis=0, keepdims=True)
        o_ref[1:2, :] = jnp.max(x_ref[...], axis=0, keepdims=True)

    return pl.pallas_call(
        body,
        out_shape=jax.ShapeDtypeStruct((2, 384), jnp.float32),
    )(pcv)


def _histogram_sc(pc_flat, params96):
    mesh = plsc.VectorSubcoreMesh(core_axis_name="c", subcore_axis_name="s")
    cp = pltpu.CompilerParams()
    if "needs_layout_passes" in pltpu.CompilerParams.__dataclass_fields__:
        cp = dataclasses.replace(cp, needs_layout_passes=False)

    @pl.kernel(
        out_type=jax.ShapeDtypeStruct((2, 2, N_BINS), jnp.float32),
        mesh=mesh,
        compiler_params=cp,
        scratch_types=[
            pltpu.VMEM((CHUNK_F,), jnp.float32),
            pltpu.VMEM((128,), jnp.int32),
            pltpu.VMEM((128,), jnp.float32),
            pltpu.VMEM((128,), jnp.float32),
            pltpu.VMEM((BINS_PER_TILE,), jnp.float32),
            pltpu.VMEM((96,), jnp.float32),
            pltpu.VMEM_SHARED((N_BINS,), jnp.float32),
            pltpu.VMEM_SHARED((N_BINS,), jnp.float32),
        ],
    )
    def k(pc_hbm, prm_hbm, out_hbm, pts, idxb, valb, onesb, zb, prm,
          hsum, hcnt):
        core = lax.axis_index("c")
        sid = lax.axis_index("s")
        wid = sid * 2 + core

        pltpu.sync_copy(prm_hbm, prm)

        ones16 = jnp.full((16,), 1.0, jnp.float32)
        zeros16 = jnp.zeros((16,), jnp.float32)

        @pl.loop(0, BINS_PER_TILE, step=16)
        def _(i):
            zb[pl.ds(i, 16)] = zeros16

        @pl.loop(0, 128, step=16)
        def _(i):
            onesb[pl.ds(i, 16)] = ones16

        pltpu.sync_copy(zb, hsum.at[pl.ds(sid * BINS_PER_TILE, BINS_PER_TILE)])
        pltpu.sync_copy(zb, hcnt.at[pl.ds(sid * BINS_PER_TILE, BINS_PER_TILE)])
        plsc.subcore_barrier()

        lane = lax.iota(jnp.int32, 16)
        lane3 = lane * 3
        xmin = prm[pl.ds(0, 16)]
        dx = prm[pl.ds(16, 16)]
        ymin = prm[pl.ds(32, 16)]
        dy = prm[pl.ds(48, 16)]
        zmin = prm[pl.ds(64, 16)]
        dz = prm[pl.ds(80, 16)]
        cw = jnp.full((16,), float(WIDTH - 1), jnp.float32)
        ch = jnp.full((16,), float(HEIGHT - 1), jnp.float32)
        ci = jnp.full((16,), INTENSITY, jnp.float32)
        max_bin = jnp.full((16,), WIDTH - 1, jnp.int32)

        @pl.loop(0, ITERS)
        def _(it):
            c = wid + it * N_WORKERS

            @pl.when(c < N_CHUNKS)
            def _():
                pltpu.sync_copy(pc_hbm.at[pl.ds(c * CHUNK_F, CHUNK_F)], pts)
                for b in range(BATCHES):
                    for j in range(8):
                        off = b * 384 + j * 48
                        gx = plsc.load_gather(pts, [lane3 + off])
                        gy = plsc.load_gather(pts, [lane3 + (off + 1)])
                        gz = plsc.load_gather(pts, [lane3 + (off + 2)])
                        vx = (cw * (gx - xmin)) / dx
                        vy = (ch * (gy - ymin)) / dy
                        vz = (ci * (gz - zmin)) / dz
                        bx = jnp.minimum(vx.astype(jnp.int32), max_bin)
                        by = jnp.minimum(vy.astype(jnp.int32), max_bin)
                        pix = (max_bin - by) * WIDTH + bx
                        idxb[pl.ds(j * 16, 16)] = pix
                        valb[pl.ds(j * 16, 16)] = vz
                    pltpu.sync_copy(valb, hsum.at[idxb], add=True)
                    pltpu.sync_copy(onesb, hcnt.at[idxb], add=True)

        plsc.subcore_barrier()
        seg = pl.ds(sid * BINS_PER_TILE, BINS_PER_TILE)
        pltpu.sync_copy(hsum.at[seg], out_hbm.at[core, 0, seg])
        pltpu.sync_copy(hcnt.at[seg], out_hbm.at[core, 1, seg])

    return k(pc_flat, params96)


def _combine_tc(parts):
    def body(p_ref, o_ref):
        s = p_ref[0, 0] + p_ref[1, 0]
        c = p_ref[0, 1] + p_ref[1, 1]
        nz = c > 0.0
        o_ref[...] = jnp.where(nz, s / jnp.where(nz, c, 1.0), 0.0)

    return pl.pallas_call(
        body,
        out_shape=jax.ShapeDtypeStruct((HEIGHT, WIDTH), jnp.float32),
    )(parts)


def kernel(pc):
    mm = _minmax_tc(pc.reshape(N_FLOATS // 384, 384))
    mins = jnp.min(mm[0].reshape(128, 3), axis=0)
    maxs = jnp.max(mm[1].reshape(128, 3), axis=0)
    params = jnp.stack([
        mins[0], maxs[0] - mins[0],
        mins[1], maxs[1] - mins[1],
        mins[2], maxs[2] - mins[2],
    ])
    params96 = jnp.broadcast_to(params[:, None], (6, 16)).reshape(96)
    parts = _histogram_sc(pc.reshape(N_FLOATS), params96)
    return _combine_tc(parts.reshape(2, 2, HEIGHT, WIDTH))


# speedup vs baseline: 1471.4406x; 16.8947x over previous
import dataclasses

import jax
import jax.numpy as jnp
from jax import lax
from jax.experimental import pallas as pl
from jax.experimental.pallas import tpu as pltpu
from jax.experimental.pallas import tpu_sc as plsc

HEIGHT = 512
WIDTH = 512
INTENSITY = 255.0

N_POINTS = 2_000_000

CHUNK_PTS = 640
N_CHUNKS = N_POINTS // CHUNK_PTS
N_WORKERS = 32
ITERS = -(-N_CHUNKS // N_WORKERS)
BATCHES = CHUNK_PTS // 128
N_BINS = HEIGHT * WIDTH
BINS_PER_TILE = N_BINS // 16


def _minmax_tc(xp, yp, zp):
    def body(x_ref, y_ref, z_ref, o_ref):
        o_ref[0:1, :] = jnp.min(x_ref[...], axis=0, keepdims=True)
        o_ref[1:2, :] = jnp.min(y_ref[...], axis=0, keepdims=True)
        o_ref[2:3, :] = jnp.min(z_ref[...], axis=0, keepdims=True)
        o_ref[3:4, :] = jnp.max(x_ref[...], axis=0, keepdims=True)
        o_ref[4:5, :] = jnp.max(y_ref[...], axis=0, keepdims=True)
        o_ref[5:6, :] = jnp.max(z_ref[...], axis=0, keepdims=True)
        o_ref[6:8, :] = jnp.zeros((2, 128), jnp.float32)

    return pl.pallas_call(
        body,
        out_shape=jax.ShapeDtypeStruct((8, 128), jnp.float32),
    )(xp, yp, zp)


def _histogram_sc(xp, yp, zp, params96):
    mesh = plsc.VectorSubcoreMesh(core_axis_name="c", subcore_axis_name="s")
    cp = pltpu.CompilerParams()
    if "needs_layout_passes" in pltpu.CompilerParams.__dataclass_fields__:
        cp = dataclasses.replace(cp, needs_layout_passes=False)

    @pl.kernel(
        out_type=jax.ShapeDtypeStruct((2, 2, N_BINS), jnp.float32),
        mesh=mesh,
        compiler_params=cp,
        scratch_types=[
            pltpu.VMEM((CHUNK_PTS,), jnp.float32),
            pltpu.VMEM((CHUNK_PTS,), jnp.float32),
            pltpu.VMEM((CHUNK_PTS,), jnp.float32),
            pltpu.VMEM((128,), jnp.int32),
            pltpu.VMEM((128,), jnp.float32),
            pltpu.VMEM((128,), jnp.float32),
            pltpu.VMEM((BINS_PER_TILE,), jnp.float32),
            pltpu.VMEM((96,), jnp.float32),
            pltpu.VMEM_SHARED((N_BINS,), jnp.float32),
            pltpu.VMEM_SHARED((N_BINS,), jnp.float32),
        ],
    )
    def k(xp_hbm, yp_hbm, zp_hbm, prm_hbm, out_hbm, ptsx, ptsy, ptsz,
          idxb, valb, onesb, zb, prm, hsum, hcnt):
        core = lax.axis_index("c")
        sid = lax.axis_index("s")
        wid = sid * 2 + core

        pltpu.sync_copy(prm_hbm, prm)

        ones16 = jnp.full((16,), 1.0, jnp.float32)
        zeros16 = jnp.zeros((16,), jnp.float32)

        @pl.loop(0, BINS_PER_TILE, step=16)
        def _(i):
            zb[pl.ds(i, 16)] = zeros16

        @pl.loop(0, 128, step=16)
        def _(i):
            onesb[pl.ds(i, 16)] = ones16

        pltpu.sync_copy(zb, hsum.at[pl.ds(sid * BINS_PER_TILE, BINS_PER_TILE)])
        pltpu.sync_copy(zb, hcnt.at[pl.ds(sid * BINS_PER_TILE, BINS_PER_TILE)])
        plsc.subcore_barrier()

        xmin = prm[pl.ds(0, 16)]
        dx = prm[pl.ds(16, 16)]
        ymin = prm[pl.ds(32, 16)]
        dy = prm[pl.ds(48, 16)]
        zmin = prm[pl.ds(64, 16)]
        dz = prm[pl.ds(80, 16)]
        cw = jnp.full((16,), float(WIDTH - 1), jnp.float32)
        ch = jnp.full((16,), float(HEIGHT - 1), jnp.float32)
        ci = jnp.full((16,), INTENSITY, jnp.float32)
        max_bin = jnp.full((16,), WIDTH - 1, jnp.int32)

        @pl.loop(0, ITERS)
        def _(it):
            c = wid + it * N_WORKERS

            @pl.when(c < N_CHUNKS)
            def _():
                base = c * CHUNK_PTS
                pltpu.sync_copy(xp_hbm.at[pl.ds(base, CHUNK_PTS)], ptsx)
                pltpu.sync_copy(yp_hbm.at[pl.ds(base, CHUNK_PTS)], ptsy)
                pltpu.sync_copy(zp_hbm.at[pl.ds(base, CHUNK_PTS)], ptsz)
                for b in range(BATCHES):
                    for j in range(8):
                        p = b * 128 + j * 16
                        gx = ptsx[pl.ds(p, 16)]
                        gy = ptsy[pl.ds(p, 16)]
                        gz = ptsz[pl.ds(p, 16)]
                        vx = (cw * (gx - xmin)) / dx
                        vy = (ch * (gy - ymin)) / dy
                        vz = (ci * (gz - zmin)) / dz
                        bx = jnp.minimum(vx.astype(jnp.int32), max_bin)
                        by = jnp.minimum(vy.astype(jnp.int32), max_bin)
                        pix = (max_bin - by) * WIDTH + bx
                        idxb[pl.ds(j * 16, 16)] = pix
                        valb[pl.ds(j * 16, 16)] = vz
                    pltpu.sync_copy(valb, hsum.at[idxb], add=True)
                    pltpu.sync_copy(onesb, hcnt.at[idxb], add=True)

        plsc.subcore_barrier()
        seg = pl.ds(sid * BINS_PER_TILE, BINS_PER_TILE)
        pltpu.sync_copy(hsum.at[seg], out_hbm.at[core, 0, seg])
        pltpu.sync_copy(hcnt.at[seg], out_hbm.at[core, 1, seg])

    return k(xp, yp, zp, params96)


def _combine_tc(parts):
    def body(p_ref, o_ref):
        s = p_ref[0, 0] + p_ref[1, 0]
        c = p_ref[0, 1] + p_ref[1, 1]
        nz = c > 0.0
        o_ref[...] = jnp.where(nz, s / jnp.where(nz, c, 1.0), 0.0)

    return pl.pallas_call(
        body,
        out_shape=jax.ShapeDtypeStruct((HEIGHT, WIDTH), jnp.float32),
    )(parts)


def kernel(pc):
    xp = pc[:, 0]
    yp = pc[:, 1]
    zp = pc[:, 2]
    mm = _minmax_tc(xp.reshape(-1, 128), yp.reshape(-1, 128),
                    zp.reshape(-1, 128))
    mins = jnp.min(mm[0:3], axis=1)
    maxs = jnp.max(mm[3:6], axis=1)
    params = jnp.stack([
        mins[0], maxs[0] - mins[0],
        mins[1], maxs[1] - mins[1],
        mins[2], maxs[2] - mins[2],
    ])
    params96 = jnp.broadcast_to(params[:, None], (6, 16)).reshape(96)
    parts = _histogram_sc(xp, yp, zp, params96)
    return _combine_tc(parts.reshape(2, 2, HEIGHT, WIDTH))


# speedup vs baseline: 3026.7976x; 2.0570x over previous
import dataclasses

import jax
import jax.numpy as jnp
from jax import lax
from jax.experimental import pallas as pl
from jax.experimental.pallas import tpu as pltpu
from jax.experimental.pallas import tpu_sc as plsc

HEIGHT = 512
WIDTH = 512
INTENSITY = 255.0

N_POINTS = 2_000_000

CHUNK_PTS = 640
N_CHUNKS = N_POINTS // CHUNK_PTS
N_WORKERS = 32
ITERS = -(-N_CHUNKS // N_WORKERS)
BATCHES = CHUNK_PTS // 128
N_BINS = HEIGHT * WIDTH
BINS_PER_TILE = N_BINS // 16


def _minmax_tc(xp, yp, zp):
    def body(x_ref, y_ref, z_ref, o_ref):
        o_ref[0:1, :] = jnp.min(x_ref[...], axis=0, keepdims=True)
        o_ref[1:2, :] = jnp.min(y_ref[...], axis=0, keepdims=True)
        o_ref[2:3, :] = jnp.min(z_ref[...], axis=0, keepdims=True)
        o_ref[3:4, :] = jnp.max(x_ref[...], axis=0, keepdims=True)
        o_ref[4:5, :] = jnp.max(y_ref[...], axis=0, keepdims=True)
        o_ref[5:6, :] = jnp.max(z_ref[...], axis=0, keepdims=True)
        o_ref[6:8, :] = jnp.zeros((2, 128), jnp.float32)

    return pl.pallas_call(
        body,
        out_shape=jax.ShapeDtypeStruct((8, 128), jnp.float32),
    )(xp, yp, zp)


def _histogram_sc(xp, yp, zp, params96):
    mesh = plsc.VectorSubcoreMesh(core_axis_name="c", subcore_axis_name="s")
    cp = pltpu.CompilerParams()
    if "needs_layout_passes" in pltpu.CompilerParams.__dataclass_fields__:
        cp = dataclasses.replace(cp, needs_layout_passes=False)

    @pl.kernel(
        out_type=jax.ShapeDtypeStruct((2, 2, N_BINS), jnp.float32),
        mesh=mesh,
        compiler_params=cp,
        scratch_types=[
            pltpu.VMEM((2, CHUNK_PTS), jnp.float32),
            pltpu.VMEM((2, CHUNK_PTS), jnp.float32),
            pltpu.VMEM((2, CHUNK_PTS), jnp.float32),
            pltpu.VMEM((2 * BATCHES, 128), jnp.int32),
            pltpu.VMEM((2 * BATCHES, 128), jnp.float32),
            pltpu.VMEM((128,), jnp.float32),
            pltpu.VMEM((BINS_PER_TILE,), jnp.float32),
            pltpu.VMEM((96,), jnp.float32),
            pltpu.VMEM_SHARED((N_BINS,), jnp.float32),
            pltpu.VMEM_SHARED((N_BINS,), jnp.float32),
            pltpu.SemaphoreType.DMA,
            pltpu.SemaphoreType.DMA,
            pltpu.SemaphoreType.DMA,
        ],
    )
    def k(xp_hbm, yp_hbm, zp_hbm, prm_hbm, out_hbm, ptsx, ptsy, ptsz,
          idxb, valb, onesb, zb, prm, hsum, hcnt, dsem0, dsem1, ssem):
        core = lax.axis_index("c")
        sid = lax.axis_index("s")
        wid = sid * 2 + core

        pltpu.sync_copy(prm_hbm, prm)

        ones16 = jnp.full((16,), 1.0, jnp.float32)
        zeros16 = jnp.zeros((16,), jnp.float32)

        @pl.loop(0, BINS_PER_TILE, step=16)
        def _(i):
            zb[pl.ds(i, 16)] = zeros16

        @pl.loop(0, 128, step=16)
        def _(i):
            onesb[pl.ds(i, 16)] = ones16

        pltpu.sync_copy(zb, hsum.at[pl.ds(sid * BINS_PER_TILE, BINS_PER_TILE)])
        pltpu.sync_copy(zb, hcnt.at[pl.ds(sid * BINS_PER_TILE, BINS_PER_TILE)])
        plsc.subcore_barrier()

        xmin = prm[pl.ds(0, 16)]
        dx = prm[pl.ds(16, 16)]
        ymin = prm[pl.ds(32, 16)]
        dy = prm[pl.ds(48, 16)]
        zmin = prm[pl.ds(64, 16)]
        dz = prm[pl.ds(80, 16)]
        cw = jnp.full((16,), float(WIDTH - 1), jnp.float32)
        ch = jnp.full((16,), float(HEIGHT - 1), jnp.float32)
        ci = jnp.full((16,), INTENSITY, jnp.float32)
        max_bin = jnp.full((16,), WIDTH - 1, jnp.int32)

        def fire_load(c, buf, sem):
            base = c * CHUNK_PTS
            pltpu.async_copy(xp_hbm.at[pl.ds(base, CHUNK_PTS)],
                             ptsx.at[buf], sem)
            pltpu.async_copy(yp_hbm.at[pl.ds(base, CHUNK_PTS)],
                             ptsy.at[buf], sem)
            pltpu.async_copy(zp_hbm.at[pl.ds(base, CHUNK_PTS)],
                             ptsz.at[buf], sem)

        def wait_load(c, buf, sem):
            base = c * CHUNK_PTS
            pltpu.make_async_copy(xp_hbm.at[pl.ds(base, CHUNK_PTS)],
                                  ptsx.at[buf], sem).wait()
            pltpu.make_async_copy(yp_hbm.at[pl.ds(base, CHUNK_PTS)],
                                  ptsy.at[buf], sem).wait()
            pltpu.make_async_copy(zp_hbm.at[pl.ds(base, CHUNK_PTS)],
                                  ptsz.at[buf], sem).wait()

        def compute_chunk(buf):
            descs = []
            for b in range(BATCHES):
                slot = buf * BATCHES + b
                for j in range(8):
                    p = b * 128 + j * 16
                    gx = ptsx[buf, pl.ds(p, 16)]
                    gy = ptsy[buf, pl.ds(p, 16)]
                    gz = ptsz[buf, pl.ds(p, 16)]
                    vx = (cw * (gx - xmin)) / dx
                    vy = (ch * (gy - ymin)) / dy
                    vz = (ci * (gz - zmin)) / dz
                    bx = jnp.minimum(vx.astype(jnp.int32), max_bin)
                    by = jnp.minimum(vy.astype(jnp.int32), max_bin)
                    pix = (max_bin - by) * WIDTH + bx
                    idxb[slot, pl.ds(j * 16, 16)] = pix
                    valb[slot, pl.ds(j * 16, 16)] = vz
                descs.append(pltpu.async_copy(
                    valb.at[slot], hsum.at[idxb.at[slot]], ssem, add=True))
                descs.append(pltpu.async_copy(
                    onesb, hcnt.at[idxb.at[slot]], ssem, add=True))
            return descs

        PAIR_ITERS = ITERS // 2
        fire_load(wid, 0, dsem0)

        @pl.loop(0, PAIR_ITERS)
        def _(it):
            c0 = wid + it * (2 * N_WORKERS)
            c1 = c0 + N_WORKERS
            c1_ok = c1 < N_CHUNKS

            @pl.when(c1_ok)
            def _():
                fire_load(c1, 1, dsem1)

            wait_load(c0, 0, dsem0)
            descs0 = compute_chunk(0)

            @pl.when(it + 1 < PAIR_ITERS)
            def _():
                fire_load(c0 + 2 * N_WORKERS, 0, dsem0)

            @pl.when(c1_ok)
            def _():
                wait_load(c1, 1, dsem1)
                descs1 = compute_chunk(1)
                for d in descs1:
                    d.wait()

            for d in descs0:
                d.wait()

        plsc.subcore_barrier()
        seg = pl.ds(sid * BINS_PER_TILE, BINS_PER_TILE)
        pltpu.sync_copy(hsum.at[seg], out_hbm.at[core, 0, seg])
        pltpu.sync_copy(hcnt.at[seg], out_hbm.at[core, 1, seg])

    return k(xp, yp, zp, params96)


def _combine_tc(parts):
    def body(p_ref, o_ref):
        s = p_ref[0, 0] + p_ref[1, 0]
        c = p_ref[0, 1] + p_ref[1, 1]
        nz = c > 0.0
        o_ref[...] = jnp.where(nz, s / jnp.where(nz, c, 1.0), 0.0)

    return pl.pallas_call(
        body,
        out_shape=jax.ShapeDtypeStruct((HEIGHT, WIDTH), jnp.float32),
    )(parts)


def kernel(pc):
    xp = pc[:, 0]
    yp = pc[:, 1]
    zp = pc[:, 2]
    mm = _minmax_tc(xp.reshape(-1, 128), yp.reshape(-1, 128),
                    zp.reshape(-1, 128))
    mins = jnp.min(mm[0:3], axis=1)
    maxs = jnp.max(mm[3:6], axis=1)
    params = jnp.stack([
        mins[0], maxs[0] - mins[0],
        mins[1], maxs[1] - mins[1],
        mins[2], maxs[2] - mins[2],
    ])
    params96 = jnp.broadcast_to(params[:, None], (6, 16)).reshape(96)
    parts = _histogram_sc(xp, yp, zp, params96)
    return _combine_tc(parts.reshape(2, 2, HEIGHT, WIDTH))


# speedup vs baseline: 3118.6386x; 1.0303x over previous
import dataclasses

import jax
import jax.numpy as jnp
from jax import lax
from jax.experimental import pallas as pl
from jax.experimental.pallas import tpu as pltpu
from jax.experimental.pallas import tpu_sc as plsc

HEIGHT = 512
WIDTH = 512
INTENSITY = 255.0

N_POINTS = 2_000_000

CHUNK_PTS = 640
N_CHUNKS = N_POINTS // CHUNK_PTS
N_WORKERS = 32
ITERS = -(-N_CHUNKS // N_WORKERS)
BATCHES = CHUNK_PTS // 128
N_BINS = HEIGHT * WIDTH
BINS_PER_TILE = N_BINS // 16


def _params_tc(xp, yp, zp):
    def body(x_ref, y_ref, z_ref, o_ref):
        xmin = jnp.min(x_ref[...])
        ymin = jnp.min(y_ref[...])
        zmin = jnp.min(z_ref[...])
        dx = jnp.max(x_ref[...]) - xmin
        dy = jnp.max(y_ref[...]) - ymin
        dz = jnp.max(z_ref[...]) - zmin
        o_ref[0:1, :] = jnp.full((1, 16), xmin, jnp.float32)
        o_ref[1:2, :] = jnp.full((1, 16), dx, jnp.float32)
        o_ref[2:3, :] = jnp.full((1, 16), ymin, jnp.float32)
        o_ref[3:4, :] = jnp.full((1, 16), dy, jnp.float32)
        o_ref[4:5, :] = jnp.full((1, 16), zmin, jnp.float32)
        o_ref[5:6, :] = jnp.full((1, 16), dz, jnp.float32)

    return pl.pallas_call(
        body,
        out_shape=jax.ShapeDtypeStruct((6, 16), jnp.float32),
    )(xp, yp, zp)


def _histogram_sc(xp, yp, zp, params96):
    mesh = plsc.VectorSubcoreMesh(core_axis_name="c", subcore_axis_name="s")
    cp = pltpu.CompilerParams()
    if "needs_layout_passes" in pltpu.CompilerParams.__dataclass_fields__:
        cp = dataclasses.replace(cp, needs_layout_passes=False)

    @pl.kernel(
        out_type=jax.ShapeDtypeStruct((2, 2, N_BINS), jnp.float32),
        mesh=mesh,
        compiler_params=cp,
        scratch_types=[
            pltpu.VMEM((2, CHUNK_PTS), jnp.float32),
            pltpu.VMEM((2, CHUNK_PTS), jnp.float32),
            pltpu.VMEM((2, CHUNK_PTS), jnp.float32),
            pltpu.VMEM((2 * BATCHES, 128), jnp.int32),
            pltpu.VMEM((2 * BATCHES, 128), jnp.float32),
            pltpu.VMEM((128,), jnp.float32),
            pltpu.VMEM((BINS_PER_TILE,), jnp.float32),
            pltpu.VMEM((96,), jnp.float32),
            pltpu.VMEM_SHARED((N_BINS,), jnp.float32),
            pltpu.VMEM_SHARED((N_BINS,), jnp.float32),
            pltpu.SemaphoreType.DMA,
            pltpu.SemaphoreType.DMA,
            pltpu.SemaphoreType.DMA,
        ],
    )
    def k(xp_hbm, yp_hbm, zp_hbm, prm_hbm, out_hbm, ptsx, ptsy, ptsz,
          idxb, valb, onesb, zb, prm, hsum, hcnt, dsem0, dsem1, ssem):
        core = lax.axis_index("c")
        sid = lax.axis_index("s")
        wid = sid * 2 + core

        pltpu.sync_copy(prm_hbm, prm)

        ones16 = jnp.full((16,), 1.0, jnp.float32)
        zeros16 = jnp.zeros((16,), jnp.float32)

        @pl.loop(0, BINS_PER_TILE, step=16)
        def _(i):
            zb[pl.ds(i, 16)] = zeros16

        @pl.loop(0, 128, step=16)
        def _(i):
            onesb[pl.ds(i, 16)] = ones16

        pltpu.sync_copy(zb, hsum.at[pl.ds(sid * BINS_PER_TILE, BINS_PER_TILE)])
        pltpu.sync_copy(zb, hcnt.at[pl.ds(sid * BINS_PER_TILE, BINS_PER_TILE)])
        plsc.subcore_barrier()

        xmin = prm[pl.ds(0, 16)]
        dx = prm[pl.ds(16, 16)]
        ymin = prm[pl.ds(32, 16)]
        dy = prm[pl.ds(48, 16)]
        zmin = prm[pl.ds(64, 16)]
        dz = prm[pl.ds(80, 16)]
        cw = jnp.full((16,), float(WIDTH - 1), jnp.float32)
        ch = jnp.full((16,), float(HEIGHT - 1), jnp.float32)
        ci = jnp.full((16,), INTENSITY, jnp.float32)
        max_bin = jnp.full((16,), WIDTH - 1, jnp.int32)

        def fire_load(c, buf, sem):
            base = c * CHUNK_PTS
            pltpu.async_copy(xp_hbm.at[pl.ds(base, CHUNK_PTS)],
                             ptsx.at[buf], sem)
            pltpu.async_copy(yp_hbm.at[pl.ds(base, CHUNK_PTS)],
                             ptsy.at[buf], sem)
            pltpu.async_copy(zp_hbm.at[pl.ds(base, CHUNK_PTS)],
                             ptsz.at[buf], sem)

        def wait_load(c, buf, sem):
            base = c * CHUNK_PTS
            pltpu.make_async_copy(xp_hbm.at[pl.ds(base, CHUNK_PTS)],
                                  ptsx.at[buf], sem).wait()
            pltpu.make_async_copy(yp_hbm.at[pl.ds(base, CHUNK_PTS)],
                                  ptsy.at[buf], sem).wait()
            pltpu.make_async_copy(zp_hbm.at[pl.ds(base, CHUNK_PTS)],
                                  ptsz.at[buf], sem).wait()

        def compute_chunk(buf):
            descs = []
            for b in range(BATCHES):
                slot = buf * BATCHES + b
                for j in range(8):
                    p = b * 128 + j * 16
                    gx = ptsx[buf, pl.ds(p, 16)]
                    gy = ptsy[buf, pl.ds(p, 16)]
                    gz = ptsz[buf, pl.ds(p, 16)]
                    vx = (cw * (gx - xmin)) / dx
                    vy = (ch * (gy - ymin)) / dy
                    vz = (ci * (gz - zmin)) / dz
                    bx = jnp.minimum(vx.astype(jnp.int32), max_bin)
                    by = jnp.minimum(vy.astype(jnp.int32), max_bin)
                    pix = (max_bin - by) * WIDTH + bx
                    idxb[slot, pl.ds(j * 16, 16)] = pix
                    valb[slot, pl.ds(j * 16, 16)] = vz
                descs.append(pltpu.async_copy(
                    valb.at[slot], hsum.at[idxb.at[slot]], ssem, add=True))
                descs.append(pltpu.async_copy(
                    onesb, hcnt.at[idxb.at[slot]], ssem, add=True))
            return descs

        PAIR_ITERS = ITERS // 2
        fire_load(wid, 0, dsem0)

        @pl.loop(0, PAIR_ITERS)
        def _(it):
            c0 = wid + it * (2 * N_WORKERS)
            c1 = c0 + N_WORKERS
            c1_ok = c1 < N_CHUNKS

            @pl.when(c1_ok)
            def _():
                fire_load(c1, 1, dsem1)

            wait_load(c0, 0, dsem0)
            descs0 = compute_chunk(0)

            @pl.when(it + 1 < PAIR_ITERS)
            def _():
                fire_load(c0 + 2 * N_WORKERS, 0, dsem0)

            @pl.when(c1_ok)
            def _():
                wait_load(c1, 1, dsem1)
                descs1 = compute_chunk(1)
                for d in descs1:
                    d.wait()

            for d in descs0:
                d.wait()

        plsc.subcore_barrier()
        seg = pl.ds(sid * BINS_PER_TILE, BINS_PER_TILE)
        pltpu.sync_copy(hsum.at[seg], out_hbm.at[core, 0, seg])
        pltpu.sync_copy(hcnt.at[seg], out_hbm.at[core, 1, seg])

    return k(xp, yp, zp, params96)


def _combine_tc(parts):
    def body(p_ref, o_ref):
        s = p_ref[0, 0] + p_ref[1, 0]
        c = p_ref[0, 1] + p_ref[1, 1]
        nz = c > 0.0
        o_ref[...] = jnp.where(nz, s / jnp.where(nz, c, 1.0), 0.0)

    return pl.pallas_call(
        body,
        out_shape=jax.ShapeDtypeStruct((HEIGHT, WIDTH), jnp.float32),
    )(parts)


def kernel(pc):
    xp = pc[:, 0]
    yp = pc[:, 1]
    zp = pc[:, 2]
    params = _params_tc(xp.reshape(-1, 128), yp.reshape(-1, 128),
                        zp.reshape(-1, 128))
    parts = _histogram_sc(xp, yp, zp, params.reshape(96))
    return _combine_tc(parts.reshape(2, 2, HEIGHT, WIDTH))
